# baseline (speedup 1.0000x reference)
import functools

import jax
import jax.numpy as jnp
from jax import lax
from jax.experimental import pallas as pl
from jax.experimental.pallas import tpu as pltpu
from jax.experimental.pallas import tpu_sc as plsc

_NC = 2
_NS = 16
_NW = _NC * _NS
_CS = 4
_NBUF = 3


def _sc_body(x_hbm, emb_hbm, out_hbm, *refs):
    S, B, D = x_hbm.shape
    rows_per_w = S // _NW
    n_chunks = rows_per_w // _CS
    nd = D // 16

    xbufs = refs[0:_NBUF]
    ebufs = refs[_NBUF:2 * _NBUF]
    obufs = refs[2 * _NBUF:3 * _NBUF]
    sxs = refs[3 * _NBUF:4 * _NBUF]
    ses = refs[4 * _NBUF:5 * _NBUF]
    sos = refs[5 * _NBUF:6 * _NBUF]

    c = lax.axis_index("c")
    s = lax.axis_index("s")
    wid = s * _NC + c
    row0 = wid * rows_per_w

    def start_in(k):
        b = k % _NBUF
        r = row0 + k * _CS
        cx = pltpu.make_async_copy(x_hbm.at[pl.ds(r, _CS)], xbufs[b], sxs[b])
        ce = pltpu.make_async_copy(emb_hbm.at[pl.ds(r, _CS)], ebufs[b], ses[b])
        cx.start()
        ce.start()
        return cx, ce

    def start_out(k):
        b = k % _NBUF
        r = row0 + k * _CS
        co = pltpu.make_async_copy(obufs[b], out_hbm.at[pl.ds(r, _CS)], sos[b])
        co.start()
        return co

    pending_in = [None] * _NBUF
    pending_out = [None] * _NBUF
    for k in range(_NBUF - 1):
        pending_in[k % _NBUF] = start_in(k)

    for k in range(n_chunks):
        b = k % _NBUF
        cx, ce = pending_in[b]
        pending_in[b] = None
        cx.wait()
        ce.wait()

        kn = k + _NBUF - 1
        if kn < n_chunks:
            pending_in[kn % _NBUF] = start_in(kn)

        if pending_out[b] is not None:
            pending_out[b].wait()
            pending_out[b] = None

        xbuf = xbufs[b]
        ebuf = ebufs[b]
        obuf = obufs[b]

        @plsc.parallel_loop(0, _CS * nd, unroll=4)
        def _(t, xbuf=xbuf, ebuf=ebuf, obuf=obuf):
            si = t // nd
            o = (t % nd) * 16
            e = ebuf[si, pl.ds(o, 16)]
            for bb in range(B):
                obuf[si, bb, pl.ds(o, 16)] = xbuf[si, bb, pl.ds(o, 16)] + e

        pending_out[b] = start_out(k)

    for b in range(_NBUF):
        if pending_out[b] is not None:
            pending_out[b].wait()


def kernel(x, emb_table):
    S, B, D = x.shape
    mesh = plsc.VectorSubcoreMesh(core_axis_name="c", subcore_axis_name="s")
    scratch = (
        [pltpu.VMEM((_CS, B, D), jnp.float32) for _ in range(_NBUF)]
        + [pltpu.VMEM((_CS, D), jnp.float32) for _ in range(_NBUF)]
        + [pltpu.VMEM((_CS, B, D), jnp.float32) for _ in range(_NBUF)]
        + [pltpu.SemaphoreType.DMA for _ in range(3 * _NBUF)]
    )
    f = functools.partial(
        pl.kernel,
        out_type=jax.ShapeDtypeStruct((S, B, D), x.dtype),
        mesh=mesh,
        scratch_types=scratch,
    )(_sc_body)
    return f(x, emb_table)


# speedup vs baseline: 1.0272x; 1.0272x over previous
import functools

import jax
import jax.numpy as jnp
from jax import lax
from jax.experimental import pallas as pl
from jax.experimental.pallas import tpu as pltpu
from jax.experimental.pallas import tpu_sc as plsc

_NC = 2
_NS = 16
_NW = _NC * _NS
_CS = 8
_NBUF = 3


def _sc_body(x_hbm, emb_hbm, out_hbm, *refs):
    S, B, D = x_hbm.shape
    rows_per_w = S // _NW
    n_chunks = rows_per_w // _CS
    nd = D // 16

    xbufs = refs[0:_NBUF]
    ebufs = refs[_NBUF:2 * _NBUF]
    sxs = refs[2 * _NBUF:3 * _NBUF]
    ses = refs[3 * _NBUF:4 * _NBUF]
    sos = refs[4 * _NBUF:5 * _NBUF]

    c = lax.axis_index("c")
    s = lax.axis_index("s")
    wid = s * _NC + c
    row0 = wid * rows_per_w

    def start_in(k, b):
        r = row0 + k * _CS
        cx = pltpu.make_async_copy(x_hbm.at[pl.ds(r, _CS)], xbufs[b], sxs[b])
        ce = pltpu.make_async_copy(emb_hbm.at[pl.ds(r, _CS)], ebufs[b], ses[b])
        cx.start()
        ce.start()
        return cx, ce

    def start_out(k, b):
        r = row0 + k * _CS
        co = pltpu.make_async_copy(xbufs[b], out_hbm.at[pl.ds(r, _CS)], sos[b])
        co.start()
        return co

    pending_in = [None] * _NBUF
    pending_out = [None] * _NBUF
    for b in range(_NBUF - 1):
        pending_in[b] = start_in(b, b)

    for k in range(n_chunks):
        b = k % _NBUF
        cx, ce = pending_in[b]
        pending_in[b] = None
        cx.wait()
        ce.wait()

        kn = k + _NBUF - 1
        if kn < n_chunks:
            nb = kn % _NBUF
            if pending_out[nb] is not None:
                pending_out[nb].wait()
                pending_out[nb] = None
            pending_in[nb] = start_in(kn, nb)

        xbuf = xbufs[b]
        ebuf = ebufs[b]

        @plsc.parallel_loop(0, _CS * nd, unroll=4)
        def _(t, xbuf=xbuf, ebuf=ebuf):
            si = t // nd
            o = (t % nd) * 16
            e = ebuf[si, pl.ds(o, 16)]
            for bb in range(B):
                plsc.addupdate(xbuf.at[si, bb, pl.ds(o, 16)], e)

        pending_out[b] = start_out(k, b)

    for b in range(_NBUF):
        if pending_out[b] is not None:
            pending_out[b].wait()


def kernel(x, emb_table):
    S, B, D = x.shape
    mesh = plsc.VectorSubcoreMesh(core_axis_name="c", subcore_axis_name="s")
    scratch = (
        [pltpu.VMEM((_CS, B, D), jnp.float32) for _ in range(_NBUF)]
        + [pltpu.VMEM((_CS, D), jnp.float32) for _ in range(_NBUF)]
        + [pltpu.SemaphoreType.DMA for _ in range(3 * _NBUF)]
    )
    f = functools.partial(
        pl.kernel,
        out_type=jax.ShapeDtypeStruct((S, B, D), x.dtype),
        mesh=mesh,
        scratch_types=scratch,
    )(_sc_body)
    return f(x, emb_table)
